# baseline (speedup 1.0000x reference)
import functools

import jax
import jax.numpy as jnp
from jax import lax
from jax.experimental import pallas as pl
from jax.experimental.pallas import tpu as pltpu
from jax.experimental.pallas import tpu_sc as plsc

ROW_TILE = 256
SUM_TILE = 512
SC_CHUNK = 8


def _router_body(x_ref, wg_ref, e_ref, w_ref, xb_ref):
    x = x_ref[...]
    xb_ref[...] = x.astype(jnp.bfloat16)
    logits = jax.lax.dot_general(
        x, wg_ref[...], (((1,), (1,)), ((), ())),
        preferred_element_type=jnp.float32)
    m = jnp.max(logits, axis=-1, keepdims=True)
    ex = jnp.exp(logits - m)
    p = ex / jnp.sum(ex, axis=-1, keepdims=True)
    num_e = p.shape[-1]
    iota = jax.lax.broadcasted_iota(jnp.int32, p.shape, 1)
    p1 = jnp.max(p, axis=-1, keepdims=True)
    i1 = jnp.min(jnp.where(p == p1, iota, num_e), axis=-1, keepdims=True)
    pm = jnp.where(iota == i1, -jnp.inf, p)
    p2 = jnp.max(pm, axis=-1, keepdims=True)
    i2 = jnp.min(jnp.where(pm == p2, iota, num_e), axis=-1, keepdims=True)
    wsum = p1 + p2
    e_ref[...] = jnp.concatenate([i1, i2], axis=1).astype(jnp.int32)
    w_ref[...] = jnp.concatenate([p1 / wsum, p2 / wsum], axis=1)


ROUTE_TILE = 512


def _route(x2d, wg):
    n, d = x2d.shape
    nb = n // ROUTE_TILE
    return pl.pallas_call(
        _router_body,
        grid=(nb,),
        in_specs=[
            pl.BlockSpec((ROUTE_TILE, d), lambda i: (i, 0)),
            pl.BlockSpec(wg.shape, lambda i: (0, 0)),
        ],
        out_specs=(
            pl.BlockSpec((ROUTE_TILE, 2), lambda i: (i, 0)),
            pl.BlockSpec((ROUTE_TILE, 2), lambda i: (i, 0)),
            pl.BlockSpec((ROUTE_TILE, d), lambda i: (i, 0)),
        ),
        out_shape=(
            jax.ShapeDtypeStruct((n, 2), jnp.int32),
            jax.ShapeDtypeStruct((n, 2), jnp.float32),
            jax.ShapeDtypeStruct((n, d), jnp.bfloat16),
        ),
    )(x2d, wg)


def _dispatch_meta(eidx, num_e, n, row_tile, num_tiles):
    e_flat = jnp.concatenate([eidx[:, 0], eidx[:, 1]])
    onehot = (e_flat[:, None] == jnp.arange(num_e, dtype=e_flat.dtype)
              ).astype(jnp.int32)
    counts = jnp.cumsum(onehot, axis=0)
    rank = jnp.take_along_axis(counts, e_flat[:, None], axis=1)[:, 0] - 1
    sizes = counts[-1]
    pad_sizes = ((sizes + row_tile - 1) // row_tile) * row_tile
    pad_offs = jnp.cumsum(pad_sizes) - pad_sizes
    pad_end = pad_offs + pad_sizes
    slot = (pad_offs[e_flat] + rank).astype(jnp.int32)
    tile_start = jnp.arange(num_tiles, dtype=jnp.int32) * row_tile
    be = jnp.clip(jnp.searchsorted(pad_end, tile_start, side='right'),
                  0, num_e - 1).astype(jnp.int32)
    vt = (tile_start < pad_end[num_e - 1]).astype(jnp.int32)
    return be, vt, slot


def _sc_gather(table, idx):
    v, d = table.shape
    btot = idx.shape[0]
    info = plsc.get_sparse_core_info()
    nw = info.num_cores * info.num_subcores
    b_per_w = btot // nw
    chunk = min(SC_CHUNK, b_per_w)
    nbuf = 4
    nchunks = b_per_w // chunk
    assert nchunks % nbuf == 0 and b_per_w % chunk == 0
    mesh = plsc.VectorSubcoreMesh(core_axis_name="c", subcore_axis_name="s")

    row_scratch = [pltpu.VMEM((chunk, d), table.dtype) for _ in range(nbuf)]

    @functools.partial(
        pl.kernel, mesh=mesh,
        out_type=jax.ShapeDtypeStruct((btot, d), table.dtype),
        scratch_types=(
            [pltpu.VMEM((b_per_w,), jnp.int32)] + row_scratch
            + [pltpu.SemaphoreType.DMA] * (2 * nbuf)
        ),
    )
    def k(table_hbm, idx_hbm, out_hbm, idx_v, *bufs_and_sems):
        bufs = bufs_and_sems[:nbuf]
        gsem = bufs_and_sems[nbuf:2 * nbuf]
        ssem = bufs_and_sems[2 * nbuf:]
        wid = lax.axis_index("s") * info.num_cores + lax.axis_index("c")
        base = wid * b_per_w
        pltpu.sync_copy(idx_hbm.at[pl.ds(base, b_per_w)], idx_v)

        @pl.loop(0, nchunks, step=nbuf)
        def _(c0):
            for j in range(nbuf):
                @pl.when(c0 > 0)
                def _():
                    pltpu.make_async_copy(
                        bufs[j], out_hbm.at[pl.ds(base, chunk)],
                        ssem[j]).wait()
                pltpu.make_async_copy(
                    table_hbm.at[idx_v.at[pl.ds((c0 + j) * chunk, chunk)]],
                    bufs[j], gsem[j]).start()
            for j in range(nbuf):
                pltpu.make_async_copy(
                    table_hbm.at[idx_v.at[pl.ds((c0 + j) * chunk, chunk)]],
                    bufs[j], gsem[j]).wait()
                pltpu.make_async_copy(
                    bufs[j],
                    out_hbm.at[pl.ds(base + (c0 + j) * chunk, chunk)],
                    ssem[j]).start()

        for j in range(nbuf):
            pltpu.make_async_copy(
                bufs[j], out_hbm.at[pl.ds(base, chunk)], ssem[j]).wait()

    return k(table, idx)


def _bf16_as_f32(a):
    n, d = a.shape
    return jax.lax.bitcast_convert_type(
        a.reshape(n, d // 2, 2), jnp.float32)


def _f32_as_bf16(a):
    n, d = a.shape
    return jax.lax.bitcast_convert_type(a, jnp.bfloat16).reshape(n, 2 * d)


def _sc_scatter_rows(x2d, slot, pad_rows):
    n, d = x2d.shape
    p = slot.shape[0]
    info = plsc.get_sparse_core_info()
    nw = info.num_cores * info.num_subcores
    b_per_w = p // nw
    chunk = min(SC_CHUNK, b_per_w)
    nbuf = 4
    nchunks = b_per_w // chunk
    assert nchunks % nbuf == 0 and b_per_w % chunk == 0 and n % b_per_w == 0
    slot2d = slot.reshape(p // chunk, chunk)
    mesh = plsc.VectorSubcoreMesh(core_axis_name="c", subcore_axis_name="s")

    row_scratch = [pltpu.VMEM((chunk, d), x2d.dtype) for _ in range(nbuf)]

    @functools.partial(
        pl.kernel, mesh=mesh,
        out_type=jax.ShapeDtypeStruct((pad_rows, d), x2d.dtype),
        scratch_types=(
            [pltpu.VMEM((nchunks, chunk), jnp.int32)] + row_scratch
            + [pltpu.SemaphoreType.DMA] * (2 * nbuf)
        ),
    )
    def k(x_hbm, idx_hbm, out_hbm, idx_v, *bufs_and_sems):
        bufs = bufs_and_sems[:nbuf]
        gsem = bufs_and_sems[nbuf:2 * nbuf]
        ssem = bufs_and_sems[2 * nbuf:]
        wid = lax.axis_index("s") * info.num_cores + lax.axis_index("c")
        base = wid * b_per_w
        src0 = lax.rem(base, n)
        pltpu.sync_copy(idx_hbm.at[pl.ds(wid * nchunks, nchunks)], idx_v)

        @pl.loop(0, nchunks, step=nbuf)
        def _(c0):
            for j in range(nbuf):
                @pl.when(c0 > 0)
                def _():
                    pltpu.make_async_copy(
                        bufs[j], out_hbm.at[idx_v.at[0]], ssem[j]).wait()
                pltpu.make_async_copy(
                    x_hbm.at[pl.ds(src0 + (c0 + j) * chunk, chunk)],
                    bufs[j], gsem[j]).start()
            for j in range(nbuf):
                pltpu.make_async_copy(
                    x_hbm.at[pl.ds(src0 + (c0 + j) * chunk, chunk)],
                    bufs[j], gsem[j]).wait()
                pltpu.make_async_copy(
                    bufs[j], out_hbm.at[idx_v.at[c0 + j]], ssem[j]).start()

        for j in range(nbuf):
            pltpu.make_async_copy(
                bufs[j], out_hbm.at[idx_v.at[0]], ssem[j]).wait()

    return k(x2d, slot2d)


def _gelu_exact(x):
    return 0.5 * x * (1.0 + jax.lax.erf(x * (2.0 ** -0.5)))


def _mlp1_body(be_ref, vt_ref, x_ref, w1_ref, b1_ref, h_ref):
    i = pl.program_id(0)

    @pl.when(vt_ref[i] == 1)
    def _():
        wb = w1_ref[0].astype(jnp.bfloat16)
        acc = jnp.dot(x_ref[...], wb, preferred_element_type=jnp.float32)
        acc = acc + b1_ref[0]
        h_ref[...] = _gelu_exact(acc).astype(h_ref.dtype)


def _mlp2_body(be_ref, vt_ref, h_ref, w2_ref, b2_ref, y_ref):
    i = pl.program_id(0)

    @pl.when(vt_ref[i] == 1)
    def _():
        wb = w2_ref[0].astype(jnp.bfloat16)
        acc = jnp.dot(h_ref[...], wb, preferred_element_type=jnp.float32)
        y_ref[...] = (acc + b2_ref[0]).astype(y_ref.dtype)


def _grouped_mlp(x_pad, w1, b1, w2, b2, be, vt, num_tiles, row_tile):
    e, d, c = w1.shape
    b1r = b1.reshape(e, 1, c)
    b2r = b2.reshape(e, 1, b2.shape[-1])
    pad = num_tiles * row_tile

    grid1 = pltpu.PrefetchScalarGridSpec(
        num_scalar_prefetch=2,
        grid=(num_tiles,),
        in_specs=[
            pl.BlockSpec((row_tile, d), lambda i, be, vt: (i, 0)),
            pl.BlockSpec((1, d, c), lambda i, be, vt: (be[i], 0, 0)),
            pl.BlockSpec((1, 1, c), lambda i, be, vt: (be[i], 0, 0)),
        ],
        out_specs=pl.BlockSpec((row_tile, c), lambda i, be, vt: (i, 0)),
    )
    h_pad = pl.pallas_call(
        _mlp1_body,
        grid_spec=grid1,
        out_shape=jax.ShapeDtypeStruct((pad, c), jnp.bfloat16),
    )(be, vt, x_pad, w1, b1r)

    c2 = w2.shape[-1]
    grid2 = pltpu.PrefetchScalarGridSpec(
        num_scalar_prefetch=2,
        grid=(num_tiles,),
        in_specs=[
            pl.BlockSpec((row_tile, c), lambda i, be, vt: (i, 0)),
            pl.BlockSpec((1, c, c2), lambda i, be, vt: (be[i], 0, 0)),
            pl.BlockSpec((1, 1, c2), lambda i, be, vt: (be[i], 0, 0)),
        ],
        out_specs=pl.BlockSpec((row_tile, c2), lambda i, be, vt: (i, 0)),
    )
    return pl.pallas_call(
        _mlp2_body,
        grid_spec=grid2,
        out_shape=jax.ShapeDtypeStruct((pad, c2), jnp.bfloat16),
    )(be, vt, h_pad, w2, b2r)


def _sum_body(g0_ref, g1_ref, w_ref, o_ref):
    w0 = w_ref[:, 0:1]
    w1 = w_ref[:, 1:2]
    o_ref[...] = (g0_ref[...].astype(jnp.float32) * w0
                  + g1_ref[...].astype(jnp.float32) * w1)


def _pair_sum(g, w, n, c):
    nb = n // SUM_TILE
    return pl.pallas_call(
        _sum_body,
        grid=(nb,),
        in_specs=[
            pl.BlockSpec((SUM_TILE, c), lambda i: (i, 0)),
            pl.BlockSpec((SUM_TILE, c), lambda i, _nb=nb: (i + _nb, 0)),
            pl.BlockSpec((SUM_TILE, 2), lambda i: (i, 0)),
        ],
        out_specs=pl.BlockSpec((SUM_TILE, c), lambda i: (i, 0)),
        out_shape=jax.ShapeDtypeStruct((n, c), jnp.float32),
    )(g, g, w)


def kernel(x_img, Wg, W1, b1, W2, b2):
    b, s, d = x_img.shape
    e, _, c = W1.shape
    n = b * s
    x2d = x_img.reshape(n, d)

    eidx, w, xb = _route(x2d, Wg)

    num_tiles = (2 * n) // ROW_TILE + e
    be, vt, slot = _dispatch_meta(eidx, e, n, ROW_TILE, num_tiles)

    x_pad = _f32_as_bf16(
        _sc_scatter_rows(_bf16_as_f32(xb), slot, num_tiles * ROW_TILE))
    y_pad = _grouped_mlp(x_pad, W1, b1, W2, b2, be, vt,
                         num_tiles, ROW_TILE)
    g = _f32_as_bf16(_sc_gather(_bf16_as_f32(y_pad), slot))
    return _pair_sum(g, w, n, c).reshape(b, s, c)


# speedup vs baseline: 3.6177x; 3.6177x over previous
import functools

import jax
import jax.numpy as jnp
from jax import lax
from jax.experimental import pallas as pl
from jax.experimental.pallas import tpu as pltpu
from jax.experimental.pallas import tpu_sc as plsc

ROW_TILE = 256
SUM_TILE = 512
SC_CHUNK = 8


def _router_body(x_ref, wg_ref, e_ref, w_ref):
    x = x_ref[...]
    logits = jax.lax.dot_general(
        x, wg_ref[...], (((1,), (1,)), ((), ())),
        preferred_element_type=jnp.float32)
    m = jnp.max(logits, axis=-1, keepdims=True)
    ex = jnp.exp(logits - m)
    p = ex / jnp.sum(ex, axis=-1, keepdims=True)
    num_e = p.shape[-1]
    iota = jax.lax.broadcasted_iota(jnp.int32, p.shape, 1)
    p1 = jnp.max(p, axis=-1, keepdims=True)
    i1 = jnp.min(jnp.where(p == p1, iota, num_e), axis=-1, keepdims=True)
    pm = jnp.where(iota == i1, -jnp.inf, p)
    p2 = jnp.max(pm, axis=-1, keepdims=True)
    i2 = jnp.min(jnp.where(pm == p2, iota, num_e), axis=-1, keepdims=True)
    wsum = p1 + p2
    e_ref[...] = jnp.concatenate([i1, i2], axis=1).astype(jnp.int32)
    w_ref[...] = jnp.concatenate([p1 / wsum, p2 / wsum], axis=1)


ROUTE_TILE = 512


def _route(x2d, wg):
    n, d = x2d.shape
    nb = n // ROUTE_TILE
    return pl.pallas_call(
        _router_body,
        grid=(nb,),
        in_specs=[
            pl.BlockSpec((ROUTE_TILE, d), lambda i: (i, 0)),
            pl.BlockSpec(wg.shape, lambda i: (0, 0)),
        ],
        out_specs=(
            pl.BlockSpec((ROUTE_TILE, 2), lambda i: (i, 0)),
            pl.BlockSpec((ROUTE_TILE, 2), lambda i: (i, 0)),
        ),
        out_shape=(
            jax.ShapeDtypeStruct((n, 2), jnp.int32),
            jax.ShapeDtypeStruct((n, 2), jnp.float32),
        ),
    )(x2d, wg)


def _dispatch_meta(eidx, num_e, n, row_tile, num_tiles):
    e_flat = jnp.concatenate([eidx[:, 0], eidx[:, 1]])
    onehot = (e_flat[:, None] == jnp.arange(num_e, dtype=e_flat.dtype)
              ).astype(jnp.int32)
    counts = jnp.cumsum(onehot, axis=0)
    rank = jnp.take_along_axis(counts, e_flat[:, None], axis=1)[:, 0] - 1
    sizes = counts[-1]
    pad_sizes = ((sizes + row_tile - 1) // row_tile) * row_tile
    pad_offs = jnp.cumsum(pad_sizes) - pad_sizes
    pad_end = pad_offs + pad_sizes
    slot = (pad_offs[e_flat] + rank).astype(jnp.int32)
    tile_start = jnp.arange(num_tiles, dtype=jnp.int32) * row_tile
    be = jnp.clip(jnp.searchsorted(pad_end, tile_start, side='right'),
                  0, num_e - 1).astype(jnp.int32)
    vt = (tile_start < pad_end[num_e - 1]).astype(jnp.int32)
    return be, vt, slot


def _sc_gather(table, idx):
    v, d = table.shape
    btot = idx.shape[0]
    info = plsc.get_sparse_core_info()
    nw = info.num_cores * info.num_subcores
    b_per_w = btot // nw
    chunk = min(SC_CHUNK, b_per_w)
    nbuf = 4
    nchunks = b_per_w // chunk
    assert nchunks % nbuf == 0 and b_per_w % chunk == 0
    mesh = plsc.VectorSubcoreMesh(core_axis_name="c", subcore_axis_name="s")

    row_scratch = [pltpu.VMEM((chunk, d), table.dtype) for _ in range(nbuf)]

    @functools.partial(
        pl.kernel, mesh=mesh,
        out_type=jax.ShapeDtypeStruct((btot, d), table.dtype),
        scratch_types=(
            [pltpu.VMEM((b_per_w,), jnp.int32)] + row_scratch
            + [pltpu.SemaphoreType.DMA] * (2 * nbuf)
        ),
    )
    def k(table_hbm, idx_hbm, out_hbm, idx_v, *bufs_and_sems):
        bufs = bufs_and_sems[:nbuf]
        gsem = bufs_and_sems[nbuf:2 * nbuf]
        ssem = bufs_and_sems[2 * nbuf:]
        wid = lax.axis_index("s") * info.num_cores + lax.axis_index("c")
        base = wid * b_per_w
        pltpu.sync_copy(idx_hbm.at[pl.ds(base, b_per_w)], idx_v)

        @pl.loop(0, nchunks, step=nbuf)
        def _(c0):
            for j in range(nbuf):
                @pl.when(c0 > 0)
                def _():
                    pltpu.make_async_copy(
                        bufs[j], out_hbm.at[pl.ds(base, chunk)],
                        ssem[j]).wait()
                pltpu.make_async_copy(
                    table_hbm.at[idx_v.at[pl.ds((c0 + j) * chunk, chunk)]],
                    bufs[j], gsem[j]).start()
            for j in range(nbuf):
                pltpu.make_async_copy(
                    table_hbm.at[idx_v.at[pl.ds((c0 + j) * chunk, chunk)]],
                    bufs[j], gsem[j]).wait()
                pltpu.make_async_copy(
                    bufs[j],
                    out_hbm.at[pl.ds(base + (c0 + j) * chunk, chunk)],
                    ssem[j]).start()

        for j in range(nbuf):
            pltpu.make_async_copy(
                bufs[j], out_hbm.at[pl.ds(base, chunk)], ssem[j]).wait()

    return k(table, idx)


def _sc_scatter_rows(x2d, slot, pad_rows):
    n, d = x2d.shape
    p = slot.shape[0]
    info = plsc.get_sparse_core_info()
    nw = info.num_cores * info.num_subcores
    b_per_w = p // nw
    chunk = min(SC_CHUNK, b_per_w)
    nbuf = 4
    nchunks = b_per_w // chunk
    assert nchunks % nbuf == 0 and b_per_w % chunk == 0 and n % b_per_w == 0
    slot2d = slot.reshape(p // chunk, chunk)
    mesh = plsc.VectorSubcoreMesh(core_axis_name="c", subcore_axis_name="s")

    row_scratch = [pltpu.VMEM((chunk, d), x2d.dtype) for _ in range(nbuf)]

    @functools.partial(
        pl.kernel, mesh=mesh,
        out_type=jax.ShapeDtypeStruct((pad_rows, d), x2d.dtype),
        scratch_types=(
            [pltpu.VMEM((nchunks, chunk), jnp.int32)] + row_scratch
            + [pltpu.SemaphoreType.DMA] * (2 * nbuf)
        ),
    )
    def k(x_hbm, idx_hbm, out_hbm, idx_v, *bufs_and_sems):
        bufs = bufs_and_sems[:nbuf]
        gsem = bufs_and_sems[nbuf:2 * nbuf]
        ssem = bufs_and_sems[2 * nbuf:]
        wid = lax.axis_index("s") * info.num_cores + lax.axis_index("c")
        base = wid * b_per_w
        src0 = lax.rem(base, n)
        pltpu.sync_copy(idx_hbm.at[pl.ds(wid * nchunks, nchunks)], idx_v)

        @pl.loop(0, nchunks, step=nbuf)
        def _(c0):
            for j in range(nbuf):
                @pl.when(c0 > 0)
                def _():
                    pltpu.make_async_copy(
                        bufs[j], out_hbm.at[idx_v.at[0]], ssem[j]).wait()
                pltpu.make_async_copy(
                    x_hbm.at[pl.ds(src0 + (c0 + j) * chunk, chunk)],
                    bufs[j], gsem[j]).start()
            for j in range(nbuf):
                pltpu.make_async_copy(
                    x_hbm.at[pl.ds(src0 + (c0 + j) * chunk, chunk)],
                    bufs[j], gsem[j]).wait()
                pltpu.make_async_copy(
                    bufs[j], out_hbm.at[idx_v.at[c0 + j]], ssem[j]).start()

        for j in range(nbuf):
            pltpu.make_async_copy(
                bufs[j], out_hbm.at[idx_v.at[0]], ssem[j]).wait()

    return k(x2d, slot2d)


def _gelu_exact(x):
    return 0.5 * x * (1.0 + jax.lax.erf(x * (2.0 ** -0.5)))


def _mlp1_body(be_ref, vt_ref, x_ref, w1_ref, b1_ref, h_ref):
    i = pl.program_id(0)

    @pl.when(vt_ref[i] == 1)
    def _():
        xb = x_ref[...].astype(jnp.bfloat16)
        wb = w1_ref[0].astype(jnp.bfloat16)
        acc = jnp.dot(xb, wb, preferred_element_type=jnp.float32)
        acc = acc + b1_ref[0]
        h_ref[...] = _gelu_exact(acc).astype(h_ref.dtype)


def _mlp2_body(be_ref, vt_ref, h_ref, w2_ref, b2_ref, y_ref):
    i = pl.program_id(0)

    @pl.when(vt_ref[i] == 1)
    def _():
        wb = w2_ref[0].astype(jnp.bfloat16)
        acc = jnp.dot(h_ref[...], wb, preferred_element_type=jnp.float32)
        y_ref[...] = (acc + b2_ref[0]).astype(y_ref.dtype)


def _grouped_mlp(x_pad, w1, b1, w2, b2, be, vt, num_tiles, row_tile):
    e, d, c = w1.shape
    b1r = b1.reshape(e, 1, c)
    b2r = b2.reshape(e, 1, b2.shape[-1])
    pad = num_tiles * row_tile

    grid1 = pltpu.PrefetchScalarGridSpec(
        num_scalar_prefetch=2,
        grid=(num_tiles,),
        in_specs=[
            pl.BlockSpec((row_tile, d), lambda i, be, vt: (i, 0)),
            pl.BlockSpec((1, d, c), lambda i, be, vt: (be[i], 0, 0)),
            pl.BlockSpec((1, 1, c), lambda i, be, vt: (be[i], 0, 0)),
        ],
        out_specs=pl.BlockSpec((row_tile, c), lambda i, be, vt: (i, 0)),
    )
    h_pad = pl.pallas_call(
        _mlp1_body,
        grid_spec=grid1,
        out_shape=jax.ShapeDtypeStruct((pad, c), jnp.bfloat16),
    )(be, vt, x_pad, w1, b1r)

    c2 = w2.shape[-1]
    grid2 = pltpu.PrefetchScalarGridSpec(
        num_scalar_prefetch=2,
        grid=(num_tiles,),
        in_specs=[
            pl.BlockSpec((row_tile, c), lambda i, be, vt: (i, 0)),
            pl.BlockSpec((1, c, c2), lambda i, be, vt: (be[i], 0, 0)),
            pl.BlockSpec((1, 1, c2), lambda i, be, vt: (be[i], 0, 0)),
        ],
        out_specs=pl.BlockSpec((row_tile, c2), lambda i, be, vt: (i, 0)),
    )
    return pl.pallas_call(
        _mlp2_body,
        grid_spec=grid2,
        out_shape=jax.ShapeDtypeStruct((pad, c2), jnp.float32),
    )(be, vt, h_pad, w2, b2r)


def _sum_body(g0_ref, g1_ref, w_ref, o_ref):
    w0 = w_ref[:, 0:1]
    w1 = w_ref[:, 1:2]
    o_ref[...] = (g0_ref[...].astype(jnp.float32) * w0
                  + g1_ref[...].astype(jnp.float32) * w1)


def _pair_sum(g, w, n, c):
    nb = n // SUM_TILE
    return pl.pallas_call(
        _sum_body,
        grid=(nb,),
        in_specs=[
            pl.BlockSpec((SUM_TILE, c), lambda i: (i, 0)),
            pl.BlockSpec((SUM_TILE, c), lambda i, _nb=nb: (i + _nb, 0)),
            pl.BlockSpec((SUM_TILE, 2), lambda i: (i, 0)),
        ],
        out_specs=pl.BlockSpec((SUM_TILE, c), lambda i: (i, 0)),
        out_shape=jax.ShapeDtypeStruct((n, c), jnp.float32),
    )(g, g, w)


def kernel(x_img, Wg, W1, b1, W2, b2):
    b, s, d = x_img.shape
    e, _, c = W1.shape
    n = b * s
    x2d = x_img.reshape(n, d)

    eidx, w = _route(x2d, Wg)

    num_tiles = (2 * n) // ROW_TILE + e
    be, vt, slot = _dispatch_meta(eidx, e, n, ROW_TILE, num_tiles)

    x_pad = _sc_scatter_rows(x2d, slot, num_tiles * ROW_TILE)
    y_pad = _grouped_mlp(x_pad, W1, b1, W2, b2, be, vt,
                         num_tiles, ROW_TILE)
    g = _sc_gather(y_pad, slot)
    return _pair_sum(g, w, n, c).reshape(b, s, c)
